# baseline (speedup 1.0000x reference)
import functools

import jax
import jax.numpy as jnp
from jax import lax
from jax.experimental import pallas as pl
from jax.experimental.pallas import tpu as pltpu
from jax.experimental.pallas import tpu_sc as plsc

N = 10000
D = 128
E = 320000

NC = 2
NS = 16
NW = NC * NS
CHUNK = 128
CHUNKS_PER_TILE = 80
NPAIR = CHUNKS_PER_TILE // 2
HALF = CHUNKS_PER_TILE // 2
NCHUNKS = NW * CHUNKS_PER_TILE
E_PAD = CHUNK * NCHUNKS
NCHUNKS_ALLOC = NCHUNKS + 2


def _linear(x, W, b):
    def body(x_ref, w_ref, b_ref, o_ref):
        o_ref[...] = lax.dot_general(
            x_ref[...], w_ref[...], (((1,), (1,)), ((), ())),
            preferred_element_type=jnp.float32,
            precision=lax.Precision.HIGHEST,
        ) + b_ref[...]

    return pl.pallas_call(
        body,
        out_shape=jax.ShapeDtypeStruct((N, D), jnp.float32),
    )(x, W, b.reshape(1, D))


def _add_partials(p):
    def body(p_ref, o_ref):
        o_ref[...] = p_ref[0] + p_ref[1]

    return pl.pallas_call(
        body,
        out_shape=jax.ShapeDtypeStruct((N, D), jnp.float32),
    )(p)


@functools.partial(
    pl.kernel,
    out_type=jax.ShapeDtypeStruct((NC, N, D), jnp.float32),
    mesh=plsc.VectorSubcoreMesh(core_axis_name="c", subcore_axis_name="s"),
    scratch_types=[
        pltpu.VMEM((2 * HALF, CHUNK), jnp.int32),
        pltpu.VMEM((HALF, CHUNK), jnp.float32),
        pltpu.VMEM((CHUNK, D), jnp.float32),
        pltpu.VMEM((CHUNK, D), jnp.float32),
        pltpu.VMEM_SHARED((N, D), jnp.float32),
        pltpu.SemaphoreType.DMA,
        pltpu.SemaphoreType.DMA,
        pltpu.SemaphoreType.DMA,
        pltpu.SemaphoreType.DMA,
    ],
)
def _sc_aggregate(support_hbm, pk_hbm, val_hbm, out_hbm,
                  idx_a, val_a, rows_a, rows_b, acc,
                  gsem_a, gsem_b, ssem_a, ssem_b):
    cid = lax.axis_index("c")
    tid = lax.axis_index("s")
    wid = tid * NC + cid

    def scale(val_v, rows_v):
        @pl.loop(0, CHUNK // 16)
        def _(j):
            v16 = val_v[pl.ds(j * 16, 16)]
            for g in range(16):
                v = v16[g]
                r = rows_v.at[j * 16 + g]
                for d in range(D // 16):
                    sl = pl.ds(d * 16, 16)
                    r[sl] = r[sl] * v

    @pl.loop(0, CHUNK)
    def _(g):
        r = rows_a.at[g]
        for d in range(D // 16):
            r[pl.ds(d * 16, 16)] = jnp.zeros((16,), jnp.float32)

    base = tid * (N // NS)
    for j in range(5):
        pltpu.sync_copy(rows_a.at[pl.ds(0, 125)],
                        acc.at[pl.ds(base + j * 125, 125)])
    plsc.subcore_barrier()

    c_base = wid * CHUNKS_PER_TILE

    def gather(j, rows_v, sem):
        return pltpu.make_async_copy(
            support_hbm.at[idx_a.at[2 * j + 1]], rows_v, sem)

    def scatter(j, rows_v, sem):
        return pltpu.make_async_copy(rows_v, acc.at[idx_a.at[2 * j]], sem)

    for h in range(CHUNKS_PER_TILE // HALF):
        pltpu.sync_copy(
            pk_hbm.at[pl.ds(2 * (c_base + h * HALF), 2 * HALF)], idx_a)
        pltpu.sync_copy(val_hbm.at[pl.ds(c_base + h * HALF, HALF)], val_a)
        gather(0, rows_a, gsem_a).start()

        @pl.loop(0, HALF // 2)
        def _(m):
            j0 = 2 * m

            gather(j0, rows_a, gsem_a).wait()

            @pl.when(m > 0)
            def _():
                scatter(j0 - 1, rows_b, ssem_b).wait()

            gather(j0 + 1, rows_b, gsem_b).start()
            scale(val_a.at[j0], rows_a)
            scatter(j0, rows_a, ssem_a).start(add=True)

            gather(j0 + 1, rows_b, gsem_b).wait()
            scatter(j0, rows_a, ssem_a).wait()

            @pl.when(m < HALF // 2 - 1)
            def _():
                gather(j0 + 2, rows_a, gsem_a).start()

            scale(val_a.at[j0 + 1], rows_b)
            scatter(j0 + 1, rows_b, ssem_b).start(add=True)

        scatter(HALF - 1, rows_b, ssem_b).wait()

    plsc.subcore_barrier()
    wb = tid * 624
    pltpu.sync_copy(acc.at[pl.ds(wb, 624)],
                    out_hbm.at[cid, pl.ds(wb, 624)])

    @pl.when(tid == 0)
    def _():
        pltpu.sync_copy(acc.at[pl.ds(16 * 624, N - 16 * 624)],
                        out_hbm.at[cid, pl.ds(16 * 624, N - 16 * 624)])


@jax.jit
def kernel(x, adj_indices, adj_values, W, b):
    support = _linear(x, W, b)

    pad = NCHUNKS_ALLOC * CHUNK - E
    row = adj_indices[0]
    col = adj_indices[1]
    pad_rows = (jnp.arange(pad, dtype=jnp.int32) * 79) % N
    packed = jnp.stack([
        jnp.concatenate([row, pad_rows]).reshape(NCHUNKS_ALLOC, CHUNK),
        jnp.concatenate([col, pad_rows]).reshape(NCHUNKS_ALLOC, CHUNK),
    ], axis=1).reshape(2 * NCHUNKS_ALLOC, CHUNK)
    vals = jnp.pad(adj_values, (0, pad)).reshape(NCHUNKS_ALLOC, CHUNK)

    partials = _sc_aggregate(support, packed, vals)
    return _add_partials(partials)


# speedup vs baseline: 1.0007x; 1.0007x over previous
import functools

import jax
import jax.numpy as jnp
from jax import lax
from jax.experimental import pallas as pl
from jax.experimental.pallas import tpu as pltpu
from jax.experimental.pallas import tpu_sc as plsc

N = 10000
D = 128
E = 320000

NC = 2
NS = 16
NW = NC * NS
CHUNK = 128
CHUNKS_PER_TILE = 80
NPAIR = CHUNKS_PER_TILE // 2
HALF = CHUNKS_PER_TILE // 2
NCHUNKS = NW * CHUNKS_PER_TILE
E_PAD = CHUNK * NCHUNKS
NCHUNKS_ALLOC = NCHUNKS + 2


def _linear(x, W, b):
    def body(x_ref, w_ref, b_ref, o_ref):
        o_ref[...] = lax.dot_general(
            x_ref[...], w_ref[...], (((1,), (1,)), ((), ())),
            preferred_element_type=jnp.float32,
            precision=lax.Precision.HIGHEST,
        ) + b_ref[...]

    return pl.pallas_call(
        body,
        out_shape=jax.ShapeDtypeStruct((N, D), jnp.float32),
    )(x, W, b.reshape(1, D))


def _add_partials(p):
    def body(p_ref, o_ref):
        o_ref[...] = p_ref[0] + p_ref[1]

    return pl.pallas_call(
        body,
        out_shape=jax.ShapeDtypeStruct((N, D), jnp.float32),
    )(p)


@functools.partial(
    pl.kernel,
    out_type=jax.ShapeDtypeStruct((NC, N, D), jnp.float32),
    mesh=plsc.VectorSubcoreMesh(core_axis_name="c", subcore_axis_name="s"),
    scratch_types=[
        pltpu.VMEM((2 * HALF, CHUNK), jnp.int32),
        pltpu.VMEM((HALF, CHUNK), jnp.float32),
        pltpu.VMEM((CHUNK, D), jnp.float32),
        pltpu.VMEM((CHUNK, D), jnp.float32),
        pltpu.VMEM_SHARED((N, D), jnp.float32),
        pltpu.SemaphoreType.DMA,
        pltpu.SemaphoreType.DMA,
        pltpu.SemaphoreType.DMA,
        pltpu.SemaphoreType.DMA,
    ],
)
def _sc_aggregate(support_hbm, pk_hbm, val_hbm, out_hbm,
                  idx_a, val_a, rows_a, rows_b, acc,
                  gsem_a, gsem_b, ssem_a, ssem_b):
    cid = lax.axis_index("c")
    tid = lax.axis_index("s")
    wid = tid * NC + cid

    def scale(val_v, rows_v):
        @plsc.parallel_loop(0, CHUNK // 16, unroll=2)
        def _(j):
            v16 = val_v[pl.ds(j * 16, 16)]
            for g in range(16):
                v = v16[g]
                r = rows_v.at[j * 16 + g]
                for d in range(D // 16):
                    sl = pl.ds(d * 16, 16)
                    r[sl] = r[sl] * v

    @pl.loop(0, CHUNK)
    def _(g):
        r = rows_a.at[g]
        for d in range(D // 16):
            r[pl.ds(d * 16, 16)] = jnp.zeros((16,), jnp.float32)

    base = tid * (N // NS)
    for j in range(5):
        pltpu.sync_copy(rows_a.at[pl.ds(0, 125)],
                        acc.at[pl.ds(base + j * 125, 125)])
    plsc.subcore_barrier()

    c_base = wid * CHUNKS_PER_TILE

    def gather(j, rows_v, sem):
        return pltpu.make_async_copy(
            support_hbm.at[idx_a.at[2 * j + 1]], rows_v, sem)

    def scatter(j, rows_v, sem):
        return pltpu.make_async_copy(rows_v, acc.at[idx_a.at[2 * j]], sem)

    for h in range(CHUNKS_PER_TILE // HALF):
        pltpu.sync_copy(
            pk_hbm.at[pl.ds(2 * (c_base + h * HALF), 2 * HALF)], idx_a)
        pltpu.sync_copy(val_hbm.at[pl.ds(c_base + h * HALF, HALF)], val_a)
        gather(0, rows_a, gsem_a).start()

        @pl.loop(0, HALF // 2)
        def _(m):
            j0 = 2 * m

            gather(j0, rows_a, gsem_a).wait()

            @pl.when(m > 0)
            def _():
                scatter(j0 - 1, rows_b, ssem_b).wait()

            gather(j0 + 1, rows_b, gsem_b).start()
            scale(val_a.at[j0], rows_a)
            scatter(j0, rows_a, ssem_a).start(add=True)

            gather(j0 + 1, rows_b, gsem_b).wait()
            scatter(j0, rows_a, ssem_a).wait()

            @pl.when(m < HALF // 2 - 1)
            def _():
                gather(j0 + 2, rows_a, gsem_a).start()

            scale(val_a.at[j0 + 1], rows_b)
            scatter(j0 + 1, rows_b, ssem_b).start(add=True)

        scatter(HALF - 1, rows_b, ssem_b).wait()

    plsc.subcore_barrier()
    wb = tid * 624
    pltpu.sync_copy(acc.at[pl.ds(wb, 624)],
                    out_hbm.at[cid, pl.ds(wb, 624)])

    @pl.when(tid == 0)
    def _():
        pltpu.sync_copy(acc.at[pl.ds(16 * 624, N - 16 * 624)],
                        out_hbm.at[cid, pl.ds(16 * 624, N - 16 * 624)])


@jax.jit
def kernel(x, adj_indices, adj_values, W, b):
    support = _linear(x, W, b)

    pad = NCHUNKS_ALLOC * CHUNK - E
    row = adj_indices[0]
    col = adj_indices[1]
    pad_rows = (jnp.arange(pad, dtype=jnp.int32) * 79) % N
    packed = jnp.stack([
        jnp.concatenate([row, pad_rows]).reshape(NCHUNKS_ALLOC, CHUNK),
        jnp.concatenate([col, pad_rows]).reshape(NCHUNKS_ALLOC, CHUNK),
    ], axis=1).reshape(2 * NCHUNKS_ALLOC, CHUNK)
    vals = jnp.pad(adj_values, (0, pad)).reshape(NCHUNKS_ALLOC, CHUNK)

    partials = _sc_aggregate(support, packed, vals)
    return _add_partials(partials)


# speedup vs baseline: 1.0059x; 1.0052x over previous
import functools

import jax
import jax.numpy as jnp
from jax import lax
from jax.experimental import pallas as pl
from jax.experimental.pallas import tpu as pltpu
from jax.experimental.pallas import tpu_sc as plsc

N = 10000
D = 128
E = 320000

NC = 2
NS = 16
NW = NC * NS
CHUNK = 128
CHUNKS_PER_TILE = 80
HALF = CHUNKS_PER_TILE // 2
NCHUNKS = NW * CHUNKS_PER_TILE
NCHUNKS_ALLOC = NCHUNKS + 2


def _linear(x, W, b):
    def body(x_ref, w_ref, b_ref, o_ref):
        o_ref[...] = lax.dot_general(
            x_ref[...], w_ref[...], (((1,), (1,)), ((), ())),
            preferred_element_type=jnp.float32,
            precision=lax.Precision.HIGHEST,
        ) + b_ref[...]

    return pl.pallas_call(
        body,
        out_shape=jax.ShapeDtypeStruct((N, D), jnp.float32),
    )(x, W, b.reshape(1, D))


def _add_partials(p):
    def body(p_ref, o_ref):
        o_ref[...] = p_ref[0] + p_ref[1]

    return pl.pallas_call(
        body,
        out_shape=jax.ShapeDtypeStruct((N, D), jnp.float32),
    )(p)


@functools.partial(
    pl.kernel,
    out_type=jax.ShapeDtypeStruct((NC, N, D), jnp.float32),
    mesh=plsc.VectorSubcoreMesh(core_axis_name="c", subcore_axis_name="s"),
    scratch_types=[
        pltpu.VMEM((2 * HALF, CHUNK), jnp.int32),
        pltpu.VMEM((HALF, CHUNK), jnp.float32),
        pltpu.VMEM((CHUNK, D), jnp.float32),
        pltpu.VMEM((CHUNK, D), jnp.float32),
        pltpu.VMEM_SHARED((N, D), jnp.float32),
        pltpu.SemaphoreType.DMA,
        pltpu.SemaphoreType.DMA,
        pltpu.SemaphoreType.DMA,
        pltpu.SemaphoreType.DMA,
    ],
)
def _sc_aggregate(support_hbm, pk_hbm, val_hbm, out_hbm,
                  idx_a, val_a, rows_a, rows_b, acc,
                  gsem_a, gsem_b, ssem_a, ssem_b):
    cid = lax.axis_index("c")
    tid = lax.axis_index("s")
    wid = tid * NC + cid

    def scale(val_v, rows_v):
        @plsc.parallel_loop(0, CHUNK // 16, unroll=2)
        def _(j):
            v16 = val_v[pl.ds(j * 16, 16)]
            for g in range(16):
                v = v16[g]
                r = rows_v.at[j * 16 + g]
                for d in range(D // 16):
                    sl = pl.ds(d * 16, 16)
                    r[sl] = r[sl] * v

    @pl.loop(0, CHUNK)
    def _(g):
        r = rows_a.at[g]
        for d in range(D // 16):
            r[pl.ds(d * 16, 16)] = jnp.zeros((16,), jnp.float32)

    base = tid * (N // NS)
    for j in range(5):
        pltpu.sync_copy(rows_a.at[pl.ds(0, 125)],
                        acc.at[pl.ds(base + j * 125, 125)])
    plsc.subcore_barrier()

    c_base = wid * CHUNKS_PER_TILE

    def gather(j, rows_v, sem):
        return pltpu.make_async_copy(
            support_hbm.at[idx_a.at[2 * j + 1]], rows_v, sem)

    def scatter(j, rows_v, sem):
        return pltpu.make_async_copy(rows_v, acc.at[idx_a.at[2 * j]], sem)

    for h in range(CHUNKS_PER_TILE // HALF):
        pltpu.sync_copy(
            pk_hbm.at[pl.ds(2 * (c_base + h * HALF), 2 * HALF)], idx_a)
        pltpu.sync_copy(val_hbm.at[pl.ds(c_base + h * HALF, HALF)], val_a)
        gather(0, rows_a, gsem_a).start()

        @pl.loop(0, HALF // 2)
        def _(m):
            j0 = 2 * m

            gather(j0, rows_a, gsem_a).wait()

            @pl.when(m > 0)
            def _():
                scatter(j0 - 1, rows_b, ssem_b).wait()

            gather(j0 + 1, rows_b, gsem_b).start()
            scale(val_a.at[j0], rows_a)
            scatter(j0, rows_a, ssem_a).start(add=True)

            gather(j0 + 1, rows_b, gsem_b).wait()
            scatter(j0, rows_a, ssem_a).wait()

            @pl.when(m < HALF // 2 - 1)
            def _():
                gather(j0 + 2, rows_a, gsem_a).start()

            scale(val_a.at[j0 + 1], rows_b)
            scatter(j0 + 1, rows_b, ssem_b).start(add=True)

        scatter(HALF - 1, rows_b, ssem_b).wait()

    plsc.subcore_barrier()
    wb = tid * 624
    pltpu.sync_copy(acc.at[pl.ds(wb, 624)],
                    out_hbm.at[cid, pl.ds(wb, 624)])

    @pl.when(tid == 0)
    def _():
        pltpu.sync_copy(acc.at[pl.ds(16 * 624, N - 16 * 624)],
                        out_hbm.at[cid, pl.ds(16 * 624, N - 16 * 624)])


@jax.jit
def kernel(x, adj_indices, adj_values, W, b):
    support = _linear(x, W, b)

    pad = NCHUNKS_ALLOC * CHUNK - E
    row = adj_indices[0]
    col = adj_indices[1]
    pad_rows = (jnp.arange(pad, dtype=jnp.int32) * 79) % N
    packed = jnp.stack([
        jnp.concatenate([row, pad_rows]).reshape(NCHUNKS_ALLOC, CHUNK),
        jnp.concatenate([col, pad_rows]).reshape(NCHUNKS_ALLOC, CHUNK),
    ], axis=1).reshape(2 * NCHUNKS_ALLOC, CHUNK)
    vals = jnp.pad(adj_values, (0, pad)).reshape(NCHUNKS_ALLOC, CHUNK)

    partials = _sc_aggregate(support, packed, vals)
    return _add_partials(partials)
